# baseline (speedup 1.0000x reference)
import functools

import jax
import jax.numpy as jnp
from jax import lax
from jax.experimental import pallas as pl
from jax.experimental.pallas import tpu as pltpu
from jax.experimental.pallas import tpu_sc as plsc

_K = 5
_BQ = 256


def _simtopk_kernel(q_ref, k_ref, idx_ref):
    q = q_ref[...]
    k = k_ref[...]
    qn = q / jnp.maximum(jnp.sqrt(jnp.sum(q * q, axis=1, keepdims=True)), 1e-12)
    kn = k / jnp.maximum(jnp.sqrt(jnp.sum(k * k, axis=1, keepdims=True)), 1e-12)
    sim = jnp.dot(qn, kn.T, preferred_element_type=jnp.float32)
    cols = jax.lax.broadcasted_iota(jnp.int32, sim.shape, 1)
    picks = []
    for _ in range(_K):
        m = jnp.max(sim, axis=1, keepdims=True)
        a = jnp.min(jnp.where(sim == m, cols, jnp.int32(2**30)), axis=1)
        picks.append(a)
        sim = jnp.where(cols == a[:, None], -jnp.inf, sim)
    idx_ref[...] = jnp.stack(picks, axis=1)


try:
    _SC_INFO = plsc.get_sparse_core_info()
    _NC, _NS = _SC_INFO.num_cores, _SC_INFO.num_subcores
except Exception:
    _NC, _NS = 2, 16
_NW = _NC * _NS
_CHUNK = 16


def _sc_gather(table_hbm, idx_hbm, out_hbm, idx_v, buf0, buf1, sem0, sem1):
    b = idx_hbm.shape[0]
    b_per_w = b // _NW
    n_chunks = b_per_w // _CHUNK
    wid = lax.axis_index("s") * _NC + lax.axis_index("c")
    base = wid * b_per_w
    pltpu.sync_copy(idx_hbm.at[pl.ds(base, b_per_w)], idx_v)
    bufs = (buf0, buf1)
    sems = (sem0, sem1)
    handles = [None] * n_chunks
    handles[0] = pltpu.async_copy(
        table_hbm.at[idx_v.at[pl.ds(0, _CHUNK)]], bufs[0], sems[0])
    for c in range(n_chunks):
        if c + 1 < n_chunks:
            handles[c + 1] = pltpu.async_copy(
                table_hbm.at[idx_v.at[pl.ds((c + 1) * _CHUNK, _CHUNK)]],
                bufs[(c + 1) % 2], sems[(c + 1) % 2])
        handles[c].wait()
        pltpu.sync_copy(bufs[c % 2], out_hbm.at[pl.ds(base + c * _CHUNK, _CHUNK)])


def kernel(query, top_k, prompts, prompt_keys):
    del top_k
    nq, d = query.shape
    n, k, _ = prompts.shape

    indices = pl.pallas_call(
        _simtopk_kernel,
        grid=(nq // _BQ,),
        in_specs=[
            pl.BlockSpec((_BQ, d), lambda i: (i, 0)),
            pl.BlockSpec((n, d), lambda i: (0, 0)),
        ],
        out_specs=pl.BlockSpec((_BQ, _K), lambda i: (i, 0)),
        out_shape=jax.ShapeDtypeStruct((nq, _K), jnp.int32),
    )(query, prompt_keys)

    b = nq * _K
    row = k * d
    gather = functools.partial(
        pl.kernel,
        out_type=jax.ShapeDtypeStruct((b, row), jnp.float32),
        mesh=plsc.VectorSubcoreMesh(core_axis_name="c", subcore_axis_name="s"),
        scratch_types=[
            pltpu.VMEM((b // _NW,), jnp.int32),
            pltpu.VMEM((_CHUNK, row), jnp.float32),
            pltpu.VMEM((_CHUNK, row), jnp.float32),
            pltpu.SemaphoreType.DMA,
            pltpu.SemaphoreType.DMA,
        ],
    )(_sc_gather)
    gathered = gather(prompts.reshape(n, row), indices.reshape(b))

    return gathered.reshape(nq, _K, k, d), indices


# speedup vs baseline: 1.3492x; 1.3492x over previous
import functools

import jax
import jax.numpy as jnp
from jax import lax
from jax.experimental import pallas as pl
from jax.experimental.pallas import tpu as pltpu
from jax.experimental.pallas import tpu_sc as plsc

_K = 5
_BQ = 256


def _simtopk_kernel(q_ref, k_ref, idx_ref):
    q = q_ref[...]
    k = k_ref[...]
    qn = q / jnp.maximum(jnp.sqrt(jnp.sum(q * q, axis=1, keepdims=True)), 1e-12)
    kn = k / jnp.maximum(jnp.sqrt(jnp.sum(k * k, axis=1, keepdims=True)), 1e-12)
    sim = jnp.dot(qn, kn.T, preferred_element_type=jnp.float32)
    cols = jax.lax.broadcasted_iota(jnp.int32, sim.shape, 1)
    picks = []
    for _ in range(_K):
        m = jnp.max(sim, axis=1, keepdims=True)
        a = jnp.min(jnp.where(sim == m, cols, jnp.int32(2**30)), axis=1)
        picks.append(a)
        sim = jnp.where(cols == a[:, None], -jnp.inf, sim)
    idx_ref[...] = jnp.stack(picks, axis=1)


try:
    _SC_INFO = plsc.get_sparse_core_info()
    _NC, _NS = _SC_INFO.num_cores, _SC_INFO.num_subcores
except Exception:
    _NC, _NS = 2, 16
_NW = _NC * _NS
_CHUNK = 8


def _sc_gather(table_hbm, idx_hbm, out_hbm, idx_v,
               buf0, buf1, gsem0, gsem1, wsem0, wsem1):
    b = idx_hbm.shape[0]
    b_per_w = b // _NW
    n_chunks = b_per_w // _CHUNK
    wid = lax.axis_index("s") * _NC + lax.axis_index("c")
    base = wid * b_per_w
    qbase = wid * (b_per_w // _K)
    pltpu.sync_copy(idx_hbm.at[pl.ds(base, b_per_w)], idx_v)
    bufs = (buf0, buf1)
    gsems = (gsem0, gsem1)
    wsems = (wsem0, wsem1)
    ghandles = [None] * n_chunks
    whandles = [[] for _ in range(n_chunks)]
    ghandles[0] = pltpu.async_copy(
        table_hbm.at[idx_v.at[pl.ds(0, _CHUNK)]], bufs[0], gsems[0])
    for c in range(n_chunks):
        p = c % 2
        ghandles[c].wait()
        for r in range(_CHUNK):
            qoff, j = divmod(c * _CHUNK + r, _K)
            whandles[c].append(pltpu.async_copy(
                bufs[p].at[r].at[pl.ds(0, _K)],
                out_hbm.at[qbase + qoff, j], wsems[p]))
        if c + 1 < n_chunks:
            if c >= 1:
                for h in whandles[c - 1]:
                    h.wait()
            ghandles[c + 1] = pltpu.async_copy(
                table_hbm.at[idx_v.at[pl.ds((c + 1) * _CHUNK, _CHUNK)]],
                bufs[(c + 1) % 2], gsems[(c + 1) % 2])
    for h in whandles[n_chunks - 2] + whandles[n_chunks - 1]:
        h.wait()


def kernel(query, top_k, prompts, prompt_keys):
    del top_k
    nq, d = query.shape
    n, k, _ = prompts.shape

    indices = pl.pallas_call(
        _simtopk_kernel,
        grid=(nq // _BQ,),
        in_specs=[
            pl.BlockSpec((_BQ, d), lambda i: (i, 0)),
            pl.BlockSpec((n, d), lambda i: (0, 0)),
        ],
        out_specs=pl.BlockSpec((_BQ, _K), lambda i: (i, 0)),
        out_shape=jax.ShapeDtypeStruct((nq, _K), jnp.int32),
    )(query, prompt_keys)

    b = nq * _K
    gather = functools.partial(
        pl.kernel,
        out_type=jax.ShapeDtypeStruct((nq, _K, k, d), jnp.float32),
        mesh=plsc.VectorSubcoreMesh(core_axis_name="c", subcore_axis_name="s"),
        scratch_types=[
            pltpu.VMEM((b // _NW,), jnp.int32),
            pltpu.VMEM((_CHUNK, 8, d), jnp.float32),
            pltpu.VMEM((_CHUNK, 8, d), jnp.float32),
            pltpu.SemaphoreType.DMA,
            pltpu.SemaphoreType.DMA,
            pltpu.SemaphoreType.DMA,
            pltpu.SemaphoreType.DMA,
        ],
    )(_sc_gather)
    table = jnp.pad(prompts, ((0, 0), (0, 8 - k), (0, 0)))
    gathered = gather(table, indices.reshape(b))

    return gathered, indices
